# baseline (speedup 1.0000x reference)
import functools

import jax
import jax.numpy as jnp
from jax.experimental import pallas as pl
from jax.experimental.pallas import tpu as pltpu

B, K, D = 16384, 8192, 256
BLK_B = 512
BLK_K = 2048


def _crd_prologue(crd_ref, crd2bf_ref, b2_ref):
    crd = crd_ref[...]
    crd2bf_ref[...] = crd.astype(jnp.bfloat16) * jnp.bfloat16(2.0)
    b2_ref[...] = jnp.sum(crd * crd, axis=1)[None, :]


def _centroid_kernel(lat_ref, crd2bf_ref, b2_ref, out_ref, minval, minidx,
                     a2_s, latbf_s):
    j = pl.program_id(1)
    nk = pl.num_programs(1)

    @pl.when(j == 0)
    def _precompute():
        lat = lat_ref[...]
        a2_s[...] = jnp.sum(lat * lat, axis=1, keepdims=True)
        latbf_s[...] = lat.astype(jnp.bfloat16)

    a2 = a2_s[...]
    ab2 = jax.lax.dot_general(
        latbf_s[...], crd2bf_ref[...], (((1,), (1,)), ((), ())),
        preferred_element_type=jnp.float32)
    d2 = (a2 + b2_ref[...]) - ab2

    local_min = jnp.min(d2, axis=1, keepdims=True)
    lane = jax.lax.broadcasted_iota(jnp.int32, d2.shape, 1) + j * BLK_K
    cand = jnp.where(d2 == local_min, lane, jnp.int32(2**31 - 1))
    local_idx = jnp.min(cand, axis=1, keepdims=True)
    local_d = jnp.sqrt(jnp.maximum(local_min, 0.0))

    @pl.when(j == 0)
    def _init():
        minval[...] = local_d.astype(jnp.bfloat16)
        minidx[...] = local_idx

    @pl.when(j != 0)
    def _update():
        m = minval[...].astype(jnp.float32)
        better = local_d < m
        minidx[...] = jnp.where(better, local_idx, minidx[...])
        minval[...] = jnp.where(better, local_d, m).astype(jnp.bfloat16)

    @pl.when(j == nk - 1)
    def _write():
        out_ref[...] = minidx[...]


@jax.jit
def kernel(latent, coords):
    crd2bf, b2 = pl.pallas_call(
        _crd_prologue,
        grid=(K // BLK_K,),
        in_specs=[pl.BlockSpec((BLK_K, D), lambda j: (j, 0))],
        out_specs=[
            pl.BlockSpec((BLK_K, D), lambda j: (j, 0)),
            pl.BlockSpec((1, BLK_K), lambda j: (0, j)),
        ],
        out_shape=[
            jax.ShapeDtypeStruct((K, D), jnp.bfloat16),
            jax.ShapeDtypeStruct((1, K), jnp.float32),
        ],
    )(coords)

    grid = (B // BLK_B, K // BLK_K)
    out = pl.pallas_call(
        _centroid_kernel,
        grid=grid,
        in_specs=[
            pl.BlockSpec((BLK_B, D), lambda i, j: (i, 0)),
            pl.BlockSpec((BLK_K, D), lambda i, j: (j, 0)),
            pl.BlockSpec((1, BLK_K), lambda i, j: (0, j)),
        ],
        out_specs=pl.BlockSpec((BLK_B, 1), lambda i, j: (i, 0)),
        out_shape=jax.ShapeDtypeStruct((B, 1), jnp.int32),
        scratch_shapes=[
            pltpu.VMEM((BLK_B, 1), jnp.bfloat16),
            pltpu.VMEM((BLK_B, 1), jnp.int32),
            pltpu.VMEM((BLK_B, 1), jnp.float32),
            pltpu.VMEM((BLK_B, D), jnp.bfloat16),
        ],
        compiler_params=pltpu.CompilerParams(
            dimension_semantics=("parallel", "arbitrary")),
    )(latent, crd2bf, b2)
    return out.reshape(B)


# speedup vs baseline: 1.1502x; 1.1502x over previous
import functools

import jax
import jax.numpy as jnp
from jax.experimental import pallas as pl
from jax.experimental.pallas import tpu as pltpu

B, K, D = 16384, 8192, 256
BLK_B = 1024
BLK_K = 2048


def _crd_prologue(crd_ref, crd2bf_ref, b2_ref):
    crd = crd_ref[...]
    crd2bf_ref[...] = crd.astype(jnp.bfloat16) * jnp.bfloat16(2.0)
    b2_ref[...] = jnp.sum(crd * crd, axis=1)[None, :]


def _centroid_kernel(lat_ref, crd2bf_ref, b2_ref, out_ref, minval, minidx):
    j = pl.program_id(1)
    nk = pl.num_programs(1)

    lat = lat_ref[...]
    a2 = jnp.sum(lat * lat, axis=1, keepdims=True)
    ab2 = jax.lax.dot_general(
        lat.astype(jnp.bfloat16), crd2bf_ref[...], (((1,), (1,)), ((), ())),
        preferred_element_type=jnp.float32)
    d2 = (a2 + b2_ref[...]) - ab2

    local_min = jnp.min(d2, axis=1, keepdims=True)
    lane = jax.lax.broadcasted_iota(jnp.int32, d2.shape, 1) + j * BLK_K
    cand = jnp.where(d2 == local_min, lane, jnp.int32(2**31 - 1))
    local_idx = jnp.min(cand, axis=1, keepdims=True)
    local_d = jnp.sqrt(jnp.maximum(local_min, 0.0))

    @pl.when(j == 0)
    def _init():
        minval[...] = local_d.astype(jnp.bfloat16)
        minidx[...] = local_idx

    @pl.when(j != 0)
    def _update():
        m = minval[...].astype(jnp.float32)
        better = local_d < m
        minidx[...] = jnp.where(better, local_idx, minidx[...])
        minval[...] = jnp.where(better, local_d, m).astype(jnp.bfloat16)

    @pl.when(j == nk - 1)
    def _write():
        out_ref[...] = minidx[...]


@jax.jit
def kernel(latent, coords):
    crd2bf, b2 = pl.pallas_call(
        _crd_prologue,
        grid=(K // BLK_K,),
        in_specs=[pl.BlockSpec((BLK_K, D), lambda j: (j, 0))],
        out_specs=[
            pl.BlockSpec((BLK_K, D), lambda j: (j, 0)),
            pl.BlockSpec((1, BLK_K), lambda j: (0, j)),
        ],
        out_shape=[
            jax.ShapeDtypeStruct((K, D), jnp.bfloat16),
            jax.ShapeDtypeStruct((1, K), jnp.float32),
        ],
    )(coords)

    grid = (B // BLK_B, K // BLK_K)
    out = pl.pallas_call(
        _centroid_kernel,
        grid=grid,
        in_specs=[
            pl.BlockSpec((BLK_B, D), lambda i, j: (i, 0)),
            pl.BlockSpec((BLK_K, D), lambda i, j: (j, 0)),
            pl.BlockSpec((1, BLK_K), lambda i, j: (0, j)),
        ],
        out_specs=pl.BlockSpec((BLK_B, 1), lambda i, j: (i, 0)),
        out_shape=jax.ShapeDtypeStruct((B, 1), jnp.int32),
        scratch_shapes=[
            pltpu.VMEM((BLK_B, 1), jnp.bfloat16),
            pltpu.VMEM((BLK_B, 1), jnp.int32),
        ],
        compiler_params=pltpu.CompilerParams(
            dimension_semantics=("parallel", "arbitrary")),
    )(latent, crd2bf, b2)
    return out.reshape(B)


# speedup vs baseline: 1.2336x; 1.0725x over previous
import functools

import jax
import jax.numpy as jnp
from jax.experimental import pallas as pl
from jax.experimental.pallas import tpu as pltpu

B, K, D = 16384, 8192, 256
BLK_B = 2048
BLK_K = 2048


def _crd_prologue(crd_ref, crd2bf_ref, b2_ref):
    crd = crd_ref[...]
    crd2bf_ref[...] = crd.astype(jnp.bfloat16) * jnp.bfloat16(2.0)
    b2_ref[...] = jnp.sum(crd * crd, axis=1)[None, :]


def _centroid_kernel(lat_ref, crd2bf_ref, b2_ref, out_ref, minval, minidx):
    j = pl.program_id(1)
    nk = pl.num_programs(1)

    lat = lat_ref[...]
    a2 = jnp.sum(lat * lat, axis=1, keepdims=True)
    ab2 = jax.lax.dot_general(
        lat.astype(jnp.bfloat16), crd2bf_ref[...], (((1,), (1,)), ((), ())),
        preferred_element_type=jnp.float32)
    d2 = (a2 + b2_ref[...]) - ab2

    local_min = jnp.min(d2, axis=1, keepdims=True)
    lane = jax.lax.broadcasted_iota(jnp.int32, d2.shape, 1) + j * BLK_K
    cand = jnp.where(d2 == local_min, lane, jnp.int32(2**31 - 1))
    local_idx = jnp.min(cand, axis=1, keepdims=True)
    local_d = jnp.sqrt(jnp.maximum(local_min, 0.0))

    @pl.when(j == 0)
    def _init():
        minval[...] = local_d.astype(jnp.bfloat16)
        minidx[...] = local_idx

    @pl.when(j != 0)
    def _update():
        m = minval[...].astype(jnp.float32)
        better = local_d < m
        minidx[...] = jnp.where(better, local_idx, minidx[...])
        minval[...] = jnp.where(better, local_d, m).astype(jnp.bfloat16)

    @pl.when(j == nk - 1)
    def _write():
        out_ref[...] = minidx[...]


@jax.jit
def kernel(latent, coords):
    crd2bf, b2 = pl.pallas_call(
        _crd_prologue,
        grid=(K // BLK_K,),
        in_specs=[pl.BlockSpec((BLK_K, D), lambda j: (j, 0))],
        out_specs=[
            pl.BlockSpec((BLK_K, D), lambda j: (j, 0)),
            pl.BlockSpec((1, BLK_K), lambda j: (0, j)),
        ],
        out_shape=[
            jax.ShapeDtypeStruct((K, D), jnp.bfloat16),
            jax.ShapeDtypeStruct((1, K), jnp.float32),
        ],
    )(coords)

    grid = (B // BLK_B, K // BLK_K)
    out = pl.pallas_call(
        _centroid_kernel,
        grid=grid,
        in_specs=[
            pl.BlockSpec((BLK_B, D), lambda i, j: (i, 0)),
            pl.BlockSpec((BLK_K, D), lambda i, j: (j, 0)),
            pl.BlockSpec((1, BLK_K), lambda i, j: (0, j)),
        ],
        out_specs=pl.BlockSpec((BLK_B, 1), lambda i, j: (i, 0)),
        out_shape=jax.ShapeDtypeStruct((B, 1), jnp.int32),
        scratch_shapes=[
            pltpu.VMEM((BLK_B, 1), jnp.bfloat16),
            pltpu.VMEM((BLK_B, 1), jnp.int32),
        ],
        compiler_params=pltpu.CompilerParams(
            dimension_semantics=("parallel", "arbitrary")),
    )(latent, crd2bf, b2)
    return out.reshape(B)


# speedup vs baseline: 1.2707x; 1.0301x over previous
import functools

import jax
import jax.numpy as jnp
from jax.experimental import pallas as pl
from jax.experimental.pallas import tpu as pltpu

B, K, D = 16384, 8192, 256
BLK_B = 4096
BLK_K = 2048


def _crd_prologue(crd_ref, crd2bf_ref, b2_ref):
    crd = crd_ref[...]
    crd2bf_ref[...] = crd.astype(jnp.bfloat16) * jnp.bfloat16(2.0)
    b2_ref[...] = jnp.sum(crd * crd, axis=1)[None, :]


def _centroid_kernel(lat_ref, crd2bf_ref, b2_ref, out_ref, minval, minidx):
    j = pl.program_id(1)
    nk = pl.num_programs(1)

    lat = lat_ref[...]
    a2 = jnp.sum(lat * lat, axis=1, keepdims=True)
    ab2 = jax.lax.dot_general(
        lat.astype(jnp.bfloat16), crd2bf_ref[...], (((1,), (1,)), ((), ())),
        preferred_element_type=jnp.float32)
    d2 = (a2 + b2_ref[...]) - ab2

    local_min = jnp.min(d2, axis=1, keepdims=True)
    lane = jax.lax.broadcasted_iota(jnp.int32, d2.shape, 1) + j * BLK_K
    cand = jnp.where(d2 == local_min, lane, jnp.int32(2**31 - 1))
    local_idx = jnp.min(cand, axis=1, keepdims=True)
    local_d = jnp.sqrt(jnp.maximum(local_min, 0.0))

    @pl.when(j == 0)
    def _init():
        minval[...] = local_d.astype(jnp.bfloat16)
        minidx[...] = local_idx

    @pl.when(j != 0)
    def _update():
        m = minval[...].astype(jnp.float32)
        better = local_d < m
        minidx[...] = jnp.where(better, local_idx, minidx[...])
        minval[...] = jnp.where(better, local_d, m).astype(jnp.bfloat16)

    @pl.when(j == nk - 1)
    def _write():
        out_ref[...] = minidx[...]


@jax.jit
def kernel(latent, coords):
    crd2bf, b2 = pl.pallas_call(
        _crd_prologue,
        grid=(K // BLK_K,),
        in_specs=[pl.BlockSpec((BLK_K, D), lambda j: (j, 0))],
        out_specs=[
            pl.BlockSpec((BLK_K, D), lambda j: (j, 0)),
            pl.BlockSpec((1, BLK_K), lambda j: (0, j)),
        ],
        out_shape=[
            jax.ShapeDtypeStruct((K, D), jnp.bfloat16),
            jax.ShapeDtypeStruct((1, K), jnp.float32),
        ],
    )(coords)

    grid = (B // BLK_B, K // BLK_K)
    out = pl.pallas_call(
        _centroid_kernel,
        grid=grid,
        in_specs=[
            pl.BlockSpec((BLK_B, D), lambda i, j: (i, 0)),
            pl.BlockSpec((BLK_K, D), lambda i, j: (j, 0)),
            pl.BlockSpec((1, BLK_K), lambda i, j: (0, j)),
        ],
        out_specs=pl.BlockSpec((BLK_B, 1), lambda i, j: (i, 0)),
        out_shape=jax.ShapeDtypeStruct((B, 1), jnp.int32),
        scratch_shapes=[
            pltpu.VMEM((BLK_B, 1), jnp.bfloat16),
            pltpu.VMEM((BLK_B, 1), jnp.int32),
        ],
        compiler_params=pltpu.CompilerParams(
            dimension_semantics=("parallel", "arbitrary")),
    )(latent, crd2bf, b2)
    return out.reshape(B)
